# Initial kernel scaffold; baseline (speedup 1.0000x reference)
import jax
import jax.numpy as jnp
from jax import lax
from jax.experimental import pallas as pl
from jax.experimental.pallas import tpu as pltpu
from jax.experimental.pallas import tpu_sc as plsc

B = 8
C = 32
N = 100000
R = 32
R3 = R * R * R
L = 16

CH1 = 2000
G1 = CH1 // L
CH2 = 10000
G2 = CH2 // L
NCH2 = N // CH2

CLIP_HI = jnp.float32(1.0 - 1e-6)
RANGE = jnp.float32(2.2)


def _voxel_index(xv, yv, zv):
    def quant(v):
        c = jnp.clip(v / RANGE + jnp.float32(0.5), jnp.float32(0.0), CLIP_HI)
        g = (c * jnp.float32(R)).astype(jnp.int32)
        return jnp.clip(g, 0, R - 1)
    return quant(xv) + R * quant(yv) + (R * R) * quant(zv)


def _body(pts_ref, feat_ref, out_ref, idx_ref, pc_ref, rc_ref,
          bins0, bins1, ibuf, f0, f1):
    cid = lax.axis_index("c")
    sid = lax.axis_index("s")
    w = cid * 16 + sid
    b = w // 4
    q = w % 4

    zeros = jnp.zeros((L,), jnp.float32)
    ones = jnp.ones((L,), jnp.float32)

    def zero_body(i, _):
        bins0[pl.ds(pl.multiple_of(i * L, L), L)] = zeros
        return 0
    lax.fori_loop(0, R3 // L, zero_body, 0)

    nch = jnp.where(q < 2, 13, 12)
    ch_start = q * 13 - jnp.maximum(q - 2, 0)

    def p1_chunk(i, _):
        n0 = (ch_start + i) * CH1
        pbase = b * (3 * N) + n0
        pltpu.sync_copy(pts_ref.at[pl.ds(pl.multiple_of(pbase, 8), CH1)],
                        f0.at[pl.ds(0, CH1)])
        pltpu.sync_copy(pts_ref.at[pl.ds(pl.multiple_of(pbase + N, 8), CH1)],
                        f1.at[pl.ds(0, CH1)])
        pltpu.sync_copy(pts_ref.at[pl.ds(pl.multiple_of(pbase + 2 * N, 8), CH1)],
                        bins1.at[pl.ds(0, CH1)])

        def p1_group(g, _):
            o = pl.multiple_of(g * L, L)
            iv = _voxel_index(f0[pl.ds(o, L)], f1[pl.ds(o, L)],
                              bins1[pl.ds(o, L)])
            ibuf[pl.ds(o, L)] = iv
            plsc.addupdate_scatter(bins0, [iv], ones)
            return 0
        lax.fori_loop(0, G1, p1_group, 0)

        dst = b * N + n0
        pltpu.sync_copy(ibuf.at[pl.ds(0, CH1)],
                        idx_ref.at[pl.ds(pl.multiple_of(dst, 8), CH1)])
        return 0
    lax.fori_loop(0, nch, p1_chunk, 0)

    pltpu.sync_copy(bins0.at[pl.ds(0, R3)],
                    pc_ref.at[pl.ds(pl.multiple_of(w * R3, 8), R3)])

    plsc.subcore_barrier()

    SUB = 2048
    for sub in range(4):
        off = q * (R3 // 4) + sub * SUB
        base = 4 * b * R3 + off
        pltpu.sync_copy(pc_ref.at[pl.ds(pl.multiple_of(base, 8), SUB)],
                        f0.at[pl.ds(0, SUB)])
        pltpu.sync_copy(pc_ref.at[pl.ds(pl.multiple_of(base + R3, 8), SUB)],
                        f1.at[pl.ds(0, SUB)])
        pltpu.sync_copy(pc_ref.at[pl.ds(pl.multiple_of(base + 2 * R3, 8), SUB)],
                        bins1.at[pl.ds(0, SUB)])
        pltpu.sync_copy(pc_ref.at[pl.ds(pl.multiple_of(base + 3 * R3, 8), SUB)],
                        bins1.at[pl.ds(SUB, SUB)])

        def rc_group(g, _):
            o = pl.multiple_of(g * L, L)
            s = (f0[pl.ds(o, L)] + f1[pl.ds(o, L)]
                 + bins1[pl.ds(o, L)] + bins1[pl.ds(SUB + o, L)])
            f0[pl.ds(o, L)] = jnp.float32(1.0) / jnp.maximum(s, jnp.float32(1.0))
            return 0
        lax.fori_loop(0, SUB // L, rc_group, 0)

        pltpu.sync_copy(f0.at[pl.ds(0, SUB)],
                        rc_ref.at[pl.ds(pl.multiple_of(b * R3 + off, 8), SUB)])

    plsc.subcore_barrier()

    for p in range(4):
        c0 = p * 8 + q * 2

        def zero2(i, _):
            o = pl.multiple_of(i * L, L)
            bins0[pl.ds(o, L)] = zeros
            bins1[pl.ds(o, L)] = zeros
            return 0
        lax.fori_loop(0, R3 // L, zero2, 0)

        def p2_chunk(ch, _):
            n0 = ch * CH2
            pltpu.sync_copy(
                idx_ref.at[pl.ds(pl.multiple_of(b * N + n0, 8), CH2)],
                ibuf.at[pl.ds(0, CH2)])
            pltpu.sync_copy(
                feat_ref.at[pl.ds(pl.multiple_of((b * C + c0) * N + n0, 8), CH2)],
                f0.at[pl.ds(0, CH2)])
            pltpu.sync_copy(
                feat_ref.at[pl.ds(pl.multiple_of((b * C + c0 + 1) * N + n0, 8), CH2)],
                f1.at[pl.ds(0, CH2)])

            def p2_group(g, _):
                o = pl.multiple_of(g * L, L)
                iv = ibuf[pl.ds(o, L)]
                plsc.addupdate_scatter(bins0, [iv], f0[pl.ds(o, L)])
                plsc.addupdate_scatter(bins1, [iv], f1[pl.ds(o, L)])
                return 0
            lax.fori_loop(0, G2, p2_group, 0)
            return 0
        lax.fori_loop(0, NCH2, p2_chunk, 0)

        def flush(sl, _):
            off = pl.multiple_of(sl * SUB, 8)
            pltpu.sync_copy(rc_ref.at[pl.ds(pl.multiple_of(b * R3, 8) + off, SUB)],
                            f0.at[pl.ds(0, SUB)])

            def scale(g, _):
                o = pl.multiple_of(g * L, L)
                r = f0[pl.ds(o, L)]
                bins0[pl.ds(off + o, L)] = bins0[pl.ds(off + o, L)] * r
                bins1[pl.ds(off + o, L)] = bins1[pl.ds(off + o, L)] * r
                return 0
            lax.fori_loop(0, SUB // L, scale, 0)

            obase = (b * C + c0) * R3 + off
            pltpu.sync_copy(bins0.at[pl.ds(off, SUB)],
                            out_ref.at[pl.ds(pl.multiple_of(obase, 8), SUB)])
            pltpu.sync_copy(bins1.at[pl.ds(off, SUB)],
                            out_ref.at[pl.ds(pl.multiple_of(obase + R3, 8), SUB)])
            return 0
        lax.fori_loop(0, R3 // SUB, flush, 0)


@jax.jit
def _grid_encode(pts_flat, feat_flat):
    mesh = plsc.VectorSubcoreMesh(core_axis_name="c", subcore_axis_name="s")
    fn = pl.kernel(
        _body,
        out_type=(
            jax.ShapeDtypeStruct((B * C * R3,), jnp.float32),
            jax.ShapeDtypeStruct((B * N,), jnp.int32),
            jax.ShapeDtypeStruct((32 * R3,), jnp.float32),
            jax.ShapeDtypeStruct((B * R3,), jnp.float32),
        ),
        mesh=mesh,
        scratch_types=[
            pltpu.VMEM((R3,), jnp.float32),
            pltpu.VMEM((R3,), jnp.float32),
            pltpu.VMEM((CH2,), jnp.int32),
            pltpu.VMEM((CH2,), jnp.float32),
            pltpu.VMEM((CH2,), jnp.float32),
        ],
    )
    return fn(pts_flat, feat_flat)


def kernel(points, feature):
    pts_flat = points.transpose(0, 2, 1).reshape(-1)
    feat_flat = feature.reshape(-1)
    grid, _, _, _ = _grid_encode(pts_flat, feat_flat)
    return grid.reshape(B, C, R, R, R)


# speedup vs baseline: 3.4807x; 3.4807x over previous
import jax
import jax.numpy as jnp
from jax import lax
from jax.experimental import pallas as pl
from jax.experimental.pallas import tpu as pltpu
from jax.experimental.pallas import tpu_sc as plsc

B = 8
C = 32
N = 100000
R = 32
R3 = R * R * R
L = 16

CH1 = 2000
G1 = CH1 // L
CH2 = 10000
G2 = CH2 // L
NCH2 = N // CH2

CLIP_HI = 1.0 - 1e-6
RANGE = 2.2


def _voxel_index(xv, yv, zv):
    def quant(v):
        c = jnp.clip(v / jnp.float32(RANGE) + jnp.float32(0.5),
                     jnp.float32(0.0), jnp.float32(CLIP_HI))
        g = (c * jnp.float32(R)).astype(jnp.int32)
        return jnp.clip(g, 0, R - 1)
    return quant(xv) + R * quant(yv) + (R * R) * quant(zv)


def _body(pts_ref, feat_ref, out_ref, idx_ref, pc_ref, rc_ref,
          bins0, bins1, ibuf, f0, f1):
    cid = lax.axis_index("c")
    sid = lax.axis_index("s")
    w = cid * 16 + sid
    b = w // 4
    q = w % 4

    zeros = jnp.zeros((L,), jnp.float32)
    ones = jnp.ones((L,), jnp.float32)

    def zero_body(i, _):
        bins0[pl.ds(pl.multiple_of(i * L, L), L)] = zeros
        return 0
    lax.fori_loop(0, R3 // L, zero_body, 0)

    nch = jnp.where(q < 2, 13, 12)
    ch_start = q * 13 - jnp.maximum(q - 2, 0)

    def p1_chunk(i, _):
        n0 = (ch_start + i) * CH1
        pbase = b * (3 * N) + n0
        pltpu.sync_copy(pts_ref.at[pl.ds(pl.multiple_of(pbase, 8), CH1)],
                        f0.at[pl.ds(0, CH1)])
        pltpu.sync_copy(pts_ref.at[pl.ds(pl.multiple_of(pbase + N, 8), CH1)],
                        f1.at[pl.ds(0, CH1)])
        pltpu.sync_copy(pts_ref.at[pl.ds(pl.multiple_of(pbase + 2 * N, 8), CH1)],
                        bins1.at[pl.ds(0, CH1)])

        def p1_group(g, _):
            o = pl.multiple_of(g * L, L)
            iv = _voxel_index(f0[pl.ds(o, L)], f1[pl.ds(o, L)],
                              bins1[pl.ds(o, L)])
            ibuf[pl.ds(o, L)] = iv
            plsc.addupdate_scatter(bins0, [iv], ones)
            return 0
        lax.fori_loop(0, G1, p1_group, 0)

        dst = b * N + n0
        pltpu.sync_copy(ibuf.at[pl.ds(0, CH1)],
                        idx_ref.at[pl.ds(pl.multiple_of(dst, 8), CH1)])
        return 0
    lax.fori_loop(0, nch, p1_chunk, 0)

    pltpu.sync_copy(bins0.at[pl.ds(0, R3)],
                    pc_ref.at[pl.ds(pl.multiple_of(w * R3, 8), R3)])

    plsc.subcore_barrier()

    SUB = 2048
    for sub in range(4):
        off = q * (R3 // 4) + sub * SUB
        base = 4 * b * R3 + off
        pltpu.sync_copy(pc_ref.at[pl.ds(pl.multiple_of(base, 8), SUB)],
                        f0.at[pl.ds(0, SUB)])
        pltpu.sync_copy(pc_ref.at[pl.ds(pl.multiple_of(base + R3, 8), SUB)],
                        f1.at[pl.ds(0, SUB)])
        pltpu.sync_copy(pc_ref.at[pl.ds(pl.multiple_of(base + 2 * R3, 8), SUB)],
                        bins1.at[pl.ds(0, SUB)])
        pltpu.sync_copy(pc_ref.at[pl.ds(pl.multiple_of(base + 3 * R3, 8), SUB)],
                        bins1.at[pl.ds(SUB, SUB)])

        def rc_group(g, _):
            o = pl.multiple_of(g * L, L)
            s = (f0[pl.ds(o, L)] + f1[pl.ds(o, L)]
                 + bins1[pl.ds(o, L)] + bins1[pl.ds(SUB + o, L)])
            f0[pl.ds(o, L)] = jnp.float32(1.0) / jnp.maximum(s, jnp.float32(1.0))
            return 0
        lax.fori_loop(0, SUB // L, rc_group, 0)

        pltpu.sync_copy(f0.at[pl.ds(0, SUB)],
                        rc_ref.at[pl.ds(pl.multiple_of(b * R3 + off, 8), SUB)])

    plsc.subcore_barrier()

    for p in range(4):
        c0 = p * 8 + q * 2

        def zero2(i, _):
            o = pl.multiple_of(i * L, L)
            bins0[pl.ds(o, L)] = zeros
            bins1[pl.ds(o, L)] = zeros
            return 0
        lax.fori_loop(0, R3 // L, zero2, 0)

        def p2_chunk(ch, _):
            n0 = ch * CH2
            pltpu.sync_copy(
                idx_ref.at[pl.ds(pl.multiple_of(b * N + n0, 8), CH2)],
                ibuf.at[pl.ds(0, CH2)])
            pltpu.sync_copy(
                feat_ref.at[pl.ds(pl.multiple_of((b * C + c0) * N + n0, 8), CH2)],
                f0.at[pl.ds(0, CH2)])
            pltpu.sync_copy(
                feat_ref.at[pl.ds(pl.multiple_of((b * C + c0 + 1) * N + n0, 8), CH2)],
                f1.at[pl.ds(0, CH2)])

            def p2_group(g, _):
                o = pl.multiple_of(g * L, L)
                iv = ibuf[pl.ds(o, L)]
                plsc.addupdate_scatter(bins0, [iv], f0[pl.ds(o, L)])
                plsc.addupdate_scatter(bins1, [iv], f1[pl.ds(o, L)])
                return 0
            lax.fori_loop(0, G2, p2_group, 0)
            return 0
        lax.fori_loop(0, NCH2, p2_chunk, 0)

        def flush(sl, _):
            off = pl.multiple_of(sl * SUB, 8)
            pltpu.sync_copy(rc_ref.at[pl.ds(pl.multiple_of(b * R3, 8) + off, SUB)],
                            f0.at[pl.ds(0, SUB)])

            def scale(g, _):
                o = pl.multiple_of(g * L, L)
                r = f0[pl.ds(o, L)]
                bins0[pl.ds(off + o, L)] = bins0[pl.ds(off + o, L)] * r
                bins1[pl.ds(off + o, L)] = bins1[pl.ds(off + o, L)] * r
                return 0
            lax.fori_loop(0, SUB // L, scale, 0)

            obase = (b * C + c0) * R3 + off
            pltpu.sync_copy(bins0.at[pl.ds(off, SUB)],
                            out_ref.at[pl.ds(pl.multiple_of(obase, 8), SUB)])
            pltpu.sync_copy(bins1.at[pl.ds(off, SUB)],
                            out_ref.at[pl.ds(pl.multiple_of(obase + R3, 8), SUB)])
            return 0
        lax.fori_loop(0, R3 // SUB, flush, 0)


@jax.jit
def _grid_encode(pts_flat, feat_flat):
    mesh = plsc.VectorSubcoreMesh(core_axis_name="c", subcore_axis_name="s")
    fn = pl.kernel(
        _body,
        out_type=(
            jax.ShapeDtypeStruct((B * C * R3,), jnp.float32),
            jax.ShapeDtypeStruct((B * N,), jnp.int32),
            jax.ShapeDtypeStruct((32 * R3,), jnp.float32),
            jax.ShapeDtypeStruct((B * R3,), jnp.float32),
        ),
        mesh=mesh,
        compiler_params=pltpu.CompilerParams(needs_layout_passes=False),
        scratch_types=[
            pltpu.VMEM((R3,), jnp.float32),
            pltpu.VMEM((R3,), jnp.float32),
            pltpu.VMEM((CH2,), jnp.int32),
            pltpu.VMEM((CH2,), jnp.float32),
            pltpu.VMEM((CH2,), jnp.float32),
        ],
    )
    return fn(pts_flat, feat_flat)


def kernel(points, feature):
    pts_flat = points.transpose(0, 2, 1).reshape(-1)
    feat_flat = feature.reshape(-1)
    grid, _, _, _ = _grid_encode(pts_flat, feat_flat)
    return grid.reshape(B, C, R, R, R)
